# baseline (speedup 1.0000x reference)
import functools

import jax
import jax.numpy as jnp
from jax import lax
from jax.experimental import pallas as pl
from jax.experimental.pallas import tpu as pltpu
from jax.experimental.pallas import tpu_sc as plsc

NN = 10000
EE = 320000
NG = 64
N_PAD = 10240
NC = 2
NS = 16
NW = NC * NS
CH = 256
NCH = -(-EE // (NW * CH))
E_PAD = NW * NCH * CH
RPT = N_PAD // NS
ZR = 128


def _sc_degree(dst_slabs, ones_rows, zero_rows):
    mesh = plsc.VectorSubcoreMesh(core_axis_name="c", subcore_axis_name="s",
                                  num_cores=NC, num_subcores=NS)

    @functools.partial(
        pl.kernel,
        out_type=jax.ShapeDtypeStruct((NC, N_PAD, 8), jnp.float32),
        mesh=mesh,
        compiler_params=pltpu.CompilerParams(use_tc_tiling_on_sc=False),
        scratch_types=[
            pltpu.VMEM((NCH, CH), jnp.int32),
            pltpu.VMEM((CH, 8), jnp.float32),
            pltpu.VMEM((ZR, 8), jnp.float32),
            pltpu.VMEM_SHARED((N_PAD, 8), jnp.float32),
            pltpu.SemaphoreType.DMA,
        ],
    )
    def k(dst_hbm, ones_hbm, z_hbm, out_hbm, dst_v, ones_v, zbuf_v, acc, sem):
        c = lax.axis_index("c")
        s = lax.axis_index("s")
        wid = c * NS + s
        pltpu.sync_copy(dst_hbm.at[wid], dst_v)
        pltpu.sync_copy(ones_hbm, ones_v)
        pltpu.sync_copy(z_hbm, zbuf_v)
        for t in range(RPT // ZR):
            pltpu.sync_copy(zbuf_v, acc.at[pl.ds(s * RPT + t * ZR, ZR)])
        plsc.subcore_barrier()

        def body(j, carry):
            pltpu.async_copy(ones_v, acc.at[dst_v.at[j]], sem, add=True)
            return carry

        lax.fori_loop(0, NCH, body, 0)

        def drain(j, carry):
            pltpu.make_async_copy(ones_v, acc.at[dst_v.at[0]], sem).wait()
            return carry

        lax.fori_loop(0, NCH, drain, 0)
        plsc.subcore_barrier()
        for t in range(RPT // ZR):
            pltpu.sync_copy(acc.at[pl.ds(s * RPT + t * ZR, ZR)], zbuf_v)
            pltpu.sync_copy(zbuf_v, out_hbm.at[c, pl.ds(s * RPT + t * ZR, ZR)])

    return k(dst_slabs, ones_rows, zero_rows)


def _sc_scatter_rows(g, src_slabs, dst_slabs, zero_rows, f):
    mesh = plsc.VectorSubcoreMesh(core_axis_name="c", subcore_axis_name="s",
                                  num_cores=NC, num_subcores=NS)

    @functools.partial(
        pl.kernel,
        out_type=jax.ShapeDtypeStruct((NC, N_PAD, f), jnp.float32),
        mesh=mesh,
        compiler_params=pltpu.CompilerParams(use_tc_tiling_on_sc=False),
        scratch_types=[
            pltpu.VMEM((NCH, CH), jnp.int32),
            pltpu.VMEM((NCH, CH), jnp.int32),
            pltpu.VMEM((2, CH, f), jnp.float32),
            pltpu.VMEM((ZR, f), jnp.float32),
            pltpu.VMEM_SHARED((N_PAD, f), jnp.float32),
            pltpu.SemaphoreType.DMA,
            pltpu.SemaphoreType.DMA,
        ],
    )
    def k(g_hbm, src_hbm, dst_hbm, z_hbm, out_hbm,
          src_v, dst_v, buf_v, zbuf_v, acc, gsem, ssem):
        c = lax.axis_index("c")
        s = lax.axis_index("s")
        wid = c * NS + s
        pltpu.sync_copy(src_hbm.at[wid], src_v)
        pltpu.sync_copy(dst_hbm.at[wid], dst_v)
        pltpu.sync_copy(z_hbm, zbuf_v)
        for t in range(RPT // ZR):
            pltpu.sync_copy(zbuf_v, acc.at[pl.ds(s * RPT + t * ZR, ZR)])
        plsc.subcore_barrier()

        pltpu.async_copy(g_hbm.at[src_v.at[0]], buf_v.at[0], gsem)

        def body(j, carry):
            pltpu.make_async_copy(g_hbm.at[src_v.at[0]], buf_v.at[0],
                                  gsem).wait()

            @pl.when(j >= 1)
            def _():
                pltpu.make_async_copy(buf_v.at[0], acc.at[dst_v.at[0]],
                                      ssem).wait()

            @pl.when(j < NCH - 1)
            def _():
                pltpu.async_copy(g_hbm.at[src_v.at[j + 1]],
                                 buf_v.at[(j + 1) % 2], gsem)

            pltpu.async_copy(buf_v.at[j % 2], acc.at[dst_v.at[j]],
                             ssem, add=True)
            return carry

        lax.fori_loop(0, NCH, body, 0)
        pltpu.make_async_copy(buf_v.at[0], acc.at[dst_v.at[0]], ssem).wait()
        plsc.subcore_barrier()
        for t in range(RPT // ZR):
            pltpu.sync_copy(acc.at[pl.ds(s * RPT + t * ZR, ZR)], zbuf_v)
            pltpu.sync_copy(zbuf_v, out_hbm.at[c, pl.ds(s * RPT + t * ZR, ZR)])

    return k(g, src_slabs, dst_slabs, zero_rows)


_BLK = 1024


def _tc_first(x_p, W1, degp):
    fo = W1.shape[1]

    def body(x_ref, w_ref, d_ref, g_ref, dv_ref):
        i = pl.program_id(0)
        deg = d_ref[0, :, 0:1] + d_ref[1, :, 0:1] + 1.0
        rows = lax.broadcasted_iota(jnp.int32, (_BLK, 1), 0) + i * _BLK
        m = (rows < NN).astype(jnp.float32)
        dinv = lax.rsqrt(deg) * m
        g_ref[...] = jnp.dot(x_ref[...], w_ref[...],
                             preferred_element_type=jnp.float32) * dinv
        dv_ref[...] = jnp.broadcast_to(dinv, (_BLK, 8))

    return pl.pallas_call(
        body,
        grid=(N_PAD // _BLK,),
        in_specs=[
            pl.BlockSpec((_BLK, 128), lambda i: (i, 0)),
            pl.BlockSpec((128, fo), lambda i: (0, 0)),
            pl.BlockSpec((NC, _BLK, 8), lambda i: (0, i, 0)),
        ],
        out_specs=[
            pl.BlockSpec((_BLK, fo), lambda i: (i, 0)),
            pl.BlockSpec((_BLK, 8), lambda i: (i, 0)),
        ],
        out_shape=[
            jax.ShapeDtypeStruct((N_PAD, fo), jnp.float32),
            jax.ShapeDtypeStruct((N_PAD, 8), jnp.float32),
        ],
    )(x_p, W1, degp)


def _tc_fuse(s2, g, dinv, b, W):
    fp = g.shape[1]
    fn = W.shape[1]

    def body(s_ref, g_ref, d_ref, b_ref, w_ref, o_ref):
        dv = d_ref[:, 0:1]
        a = jnp.maximum(dv * (s_ref[0] + s_ref[1] + g_ref[...]) + b_ref[...], 0.0)
        o_ref[...] = jnp.dot(a, w_ref[...],
                             preferred_element_type=jnp.float32) * dv

    return pl.pallas_call(
        body,
        grid=(N_PAD // _BLK,),
        in_specs=[
            pl.BlockSpec((NC, _BLK, fp), lambda i: (0, i, 0)),
            pl.BlockSpec((_BLK, fp), lambda i: (i, 0)),
            pl.BlockSpec((_BLK, 8), lambda i: (i, 0)),
            pl.BlockSpec((1, fp), lambda i: (0, 0)),
            pl.BlockSpec((fp, fn), lambda i: (0, 0)),
        ],
        out_specs=pl.BlockSpec((_BLK, fn), lambda i: (i, 0)),
        out_shape=jax.ShapeDtypeStruct((N_PAD, fn), jnp.float32),
    )(s2, g, dinv, b, W)


def _tc_pool(s2, g, dinv, b, batch_row, Wo, bo):
    def body(s_ref, g_ref, d_ref, b_ref, bt_ref, wo_ref, bo_ref, o_ref):
        dv = d_ref[0:NN, 0:1]
        h = jnp.maximum(
            dv * (s_ref[0, 0:NN] + s_ref[1, 0:NN] + g_ref[0:NN]) + b_ref[...],
            0.0)
        seg = lax.broadcasted_iota(jnp.int32, (NG, NN), 0)
        m = (seg == bt_ref[...]).astype(jnp.float32)
        sums = jnp.dot(m, h, preferred_element_type=jnp.float32)
        cnt = jnp.sum(m, axis=1, keepdims=True)
        pooled = sums / jnp.maximum(cnt, 1.0)
        o_ref[...] = jnp.dot(pooled, wo_ref[...],
                             preferred_element_type=jnp.float32) + bo_ref[...]

    return pl.pallas_call(
        body,
        out_shape=jax.ShapeDtypeStruct((NG, 1), jnp.float32),
    )(s2, g, dinv, b, batch_row, Wo, bo)


def kernel(x, edge_index, batch, W1, b1, W2, b2, W3, b3, W4, b4, Wo, bo):
    f32 = jnp.float32
    x_p = jnp.zeros((N_PAD, 128), f32).at[:NN].set(x)
    pad = E_PAD - EE
    src_p = jnp.concatenate(
        [edge_index[0], jnp.full((pad,), NN, jnp.int32)]).reshape(NW, NCH, CH)
    dst_p = jnp.concatenate(
        [edge_index[1], jnp.full((pad,), NN, jnp.int32)]).reshape(NW, NCH, CH)

    degp = _sc_degree(dst_p, jnp.ones((CH, 8), f32), jnp.zeros((ZR, 8), f32))
    g1, dinv = _tc_first(x_p, W1, degp)
    s1 = _sc_scatter_rows(g1, src_p, dst_p, jnp.zeros((ZR, 64), f32), 64)
    g2 = _tc_fuse(s1, g1, dinv, b1.reshape(1, -1), W2)
    s2 = _sc_scatter_rows(g2, src_p, dst_p, jnp.zeros((ZR, 32), f32), 32)
    g3 = _tc_fuse(s2, g2, dinv, b2.reshape(1, -1), W3)
    s3 = _sc_scatter_rows(g3, src_p, dst_p, jnp.zeros((ZR, 16), f32), 16)
    W4p = jnp.zeros((16, 8), f32).at[:, :4].set(W4)
    b4p = jnp.zeros((8,), f32).at[:4].set(b4)
    Wop = jnp.zeros((8, 1), f32).at[:4].set(Wo)
    g4 = _tc_fuse(s3, g3, dinv, b3.reshape(1, -1), W4p)
    s4 = _sc_scatter_rows(g4, src_p, dst_p, jnp.zeros((ZR, 8), f32), 8)
    return _tc_pool(s4, g4, dinv, b4p.reshape(1, -1),
                    batch.reshape(1, NN), Wop, bo.reshape(1, 1))


# speedup vs baseline: 1.2486x; 1.2486x over previous
import functools

import jax
import jax.numpy as jnp
from jax import lax
from jax.experimental import pallas as pl
from jax.experimental.pallas import tpu as pltpu
from jax.experimental.pallas import tpu_sc as plsc

NN = 10000
EE = 320000
NG = 64
N_PAD = 10240
NC = 2
NS = 16
NW = NC * NS
CH = 128
NCH = -(-EE // (NW * CH))
E_PAD = NW * NCH * CH
RPT = N_PAD // NS
ZR = 128


def _sc_degree(dst_slabs, ones_rows, zero_rows):
    mesh = plsc.VectorSubcoreMesh(core_axis_name="c", subcore_axis_name="s",
                                  num_cores=NC, num_subcores=NS)

    @functools.partial(
        pl.kernel,
        out_type=jax.ShapeDtypeStruct((NC, N_PAD, 8), jnp.float32),
        mesh=mesh,
        compiler_params=pltpu.CompilerParams(use_tc_tiling_on_sc=False),
        scratch_types=[
            pltpu.VMEM((NCH, CH), jnp.int32),
            pltpu.VMEM((CH, 8), jnp.float32),
            pltpu.VMEM((ZR, 8), jnp.float32),
            pltpu.VMEM_SHARED((N_PAD, 8), jnp.float32),
            pltpu.SemaphoreType.DMA,
        ],
    )
    def k(dst_hbm, ones_hbm, z_hbm, out_hbm, dst_v, ones_v, zbuf_v, acc, sem):
        c = lax.axis_index("c")
        s = lax.axis_index("s")
        wid = c * NS + s
        pltpu.sync_copy(dst_hbm.at[wid], dst_v)
        pltpu.sync_copy(ones_hbm, ones_v)
        pltpu.sync_copy(z_hbm, zbuf_v)
        for t in range(RPT // ZR):
            pltpu.sync_copy(zbuf_v, acc.at[pl.ds(s * RPT + t * ZR, ZR)])
        plsc.subcore_barrier()

        def body(j, carry):
            pltpu.async_copy(ones_v, acc.at[dst_v.at[j]], sem, add=True)
            return carry

        lax.fori_loop(0, NCH, body, 0)

        def drain(j, carry):
            pltpu.make_async_copy(ones_v, acc.at[dst_v.at[0]], sem).wait()
            return carry

        lax.fori_loop(0, NCH, drain, 0)
        plsc.subcore_barrier()
        for t in range(RPT // ZR):
            pltpu.sync_copy(acc.at[pl.ds(s * RPT + t * ZR, ZR)], zbuf_v)
            pltpu.sync_copy(zbuf_v, out_hbm.at[c, pl.ds(s * RPT + t * ZR, ZR)])

    return k(dst_slabs, ones_rows, zero_rows)


def _sc_scatter_rows(g, src_slabs, dst_slabs, zero_rows, f):
    mesh = plsc.VectorSubcoreMesh(core_axis_name="c", subcore_axis_name="s",
                                  num_cores=NC, num_subcores=NS)

    @functools.partial(
        pl.kernel,
        out_type=jax.ShapeDtypeStruct((NC, N_PAD, f), jnp.float32),
        mesh=mesh,
        compiler_params=pltpu.CompilerParams(use_tc_tiling_on_sc=False),
        scratch_types=[
            pltpu.VMEM((NCH, CH), jnp.int32),
            pltpu.VMEM((NCH, CH), jnp.int32),
            pltpu.VMEM((2, CH, f), jnp.float32),
            pltpu.VMEM((ZR, f), jnp.float32),
            pltpu.VMEM_SHARED((N_PAD, f), jnp.float32),
            pltpu.SemaphoreType.DMA,
            pltpu.SemaphoreType.DMA,
        ],
    )
    def k(g_hbm, src_hbm, dst_hbm, z_hbm, out_hbm,
          src_v, dst_v, buf_v, zbuf_v, acc, gsem, ssem):
        c = lax.axis_index("c")
        s = lax.axis_index("s")
        wid = c * NS + s
        pltpu.sync_copy(src_hbm.at[wid], src_v)
        pltpu.sync_copy(dst_hbm.at[wid], dst_v)
        pltpu.sync_copy(z_hbm, zbuf_v)
        for t in range(RPT // ZR):
            pltpu.sync_copy(zbuf_v, acc.at[pl.ds(s * RPT + t * ZR, ZR)])
        plsc.subcore_barrier()

        pltpu.async_copy(g_hbm.at[src_v.at[0]], buf_v.at[0], gsem)

        def body(j, carry):
            pltpu.make_async_copy(g_hbm.at[src_v.at[0]], buf_v.at[0],
                                  gsem).wait()

            @pl.when(j >= 1)
            def _():
                pltpu.make_async_copy(buf_v.at[0], acc.at[dst_v.at[0]],
                                      ssem).wait()

            @pl.when(j < NCH - 1)
            def _():
                pltpu.async_copy(g_hbm.at[src_v.at[j + 1]],
                                 buf_v.at[(j + 1) % 2], gsem)

            pltpu.async_copy(buf_v.at[j % 2], acc.at[dst_v.at[j]],
                             ssem, add=True)
            return carry

        lax.fori_loop(0, NCH, body, 0)
        pltpu.make_async_copy(buf_v.at[0], acc.at[dst_v.at[0]], ssem).wait()
        plsc.subcore_barrier()
        for t in range(RPT // ZR):
            pltpu.sync_copy(acc.at[pl.ds(s * RPT + t * ZR, ZR)], zbuf_v)
            pltpu.sync_copy(zbuf_v, out_hbm.at[c, pl.ds(s * RPT + t * ZR, ZR)])

    return k(g, src_slabs, dst_slabs, zero_rows)


_BLK = 1024


def _tc_first(x_p, W1, degp):
    fo = W1.shape[1]

    def body(x_ref, w_ref, d_ref, g_ref, dv_ref):
        i = pl.program_id(0)
        deg = d_ref[0, :, 0:1] + d_ref[1, :, 0:1] + 1.0
        rows = lax.broadcasted_iota(jnp.int32, (_BLK, 1), 0) + i * _BLK
        m = (rows < NN).astype(jnp.float32)
        dinv = lax.rsqrt(deg) * m
        g_ref[...] = jnp.dot(x_ref[...], w_ref[...],
                             preferred_element_type=jnp.float32) * dinv
        dv_ref[...] = jnp.broadcast_to(dinv, (_BLK, 8))

    return pl.pallas_call(
        body,
        grid=(N_PAD // _BLK,),
        in_specs=[
            pl.BlockSpec((_BLK, 128), lambda i: (i, 0)),
            pl.BlockSpec((128, fo), lambda i: (0, 0)),
            pl.BlockSpec((NC, _BLK, 8), lambda i: (0, i, 0)),
        ],
        out_specs=[
            pl.BlockSpec((_BLK, fo), lambda i: (i, 0)),
            pl.BlockSpec((_BLK, 8), lambda i: (i, 0)),
        ],
        out_shape=[
            jax.ShapeDtypeStruct((N_PAD, fo), jnp.float32),
            jax.ShapeDtypeStruct((N_PAD, 8), jnp.float32),
        ],
    )(x_p, W1, degp)


def _tc_fuse(s2, g, dinv, b, W):
    fp = g.shape[1]
    fn = W.shape[1]

    def body(s_ref, g_ref, d_ref, b_ref, w_ref, o_ref):
        dv = d_ref[:, 0:1]
        a = jnp.maximum(dv * (s_ref[0] + s_ref[1] + g_ref[...]) + b_ref[...], 0.0)
        o_ref[...] = jnp.dot(a, w_ref[...],
                             preferred_element_type=jnp.float32) * dv

    return pl.pallas_call(
        body,
        grid=(N_PAD // _BLK,),
        in_specs=[
            pl.BlockSpec((NC, _BLK, fp), lambda i: (0, i, 0)),
            pl.BlockSpec((_BLK, fp), lambda i: (i, 0)),
            pl.BlockSpec((_BLK, 8), lambda i: (i, 0)),
            pl.BlockSpec((1, fp), lambda i: (0, 0)),
            pl.BlockSpec((fp, fn), lambda i: (0, 0)),
        ],
        out_specs=pl.BlockSpec((_BLK, fn), lambda i: (i, 0)),
        out_shape=jax.ShapeDtypeStruct((N_PAD, fn), jnp.float32),
    )(s2, g, dinv, b, W)


def _tc_pool(s2, g, dinv, b, batch_row, Wo, bo):
    def body(s_ref, g_ref, d_ref, b_ref, bt_ref, wo_ref, bo_ref, o_ref):
        dv = d_ref[0:NN, 0:1]
        h = jnp.maximum(
            dv * (s_ref[0, 0:NN] + s_ref[1, 0:NN] + g_ref[0:NN]) + b_ref[...],
            0.0)
        seg = lax.broadcasted_iota(jnp.int32, (NG, NN), 0)
        m = (seg == bt_ref[...]).astype(jnp.float32)
        sums = jnp.dot(m, h, preferred_element_type=jnp.float32)
        cnt = jnp.sum(m, axis=1, keepdims=True)
        pooled = sums / jnp.maximum(cnt, 1.0)
        o_ref[...] = jnp.dot(pooled, wo_ref[...],
                             preferred_element_type=jnp.float32) + bo_ref[...]

    return pl.pallas_call(
        body,
        out_shape=jax.ShapeDtypeStruct((NG, 1), jnp.float32),
    )(s2, g, dinv, b, batch_row, Wo, bo)


def kernel(x, edge_index, batch, W1, b1, W2, b2, W3, b3, W4, b4, Wo, bo):
    f32 = jnp.float32
    x_p = jnp.zeros((N_PAD, 128), f32).at[:NN].set(x)
    pad = E_PAD - EE
    src_p = jnp.concatenate(
        [edge_index[0], jnp.full((pad,), NN, jnp.int32)]).reshape(NW, NCH, CH)
    dst_p = jnp.concatenate(
        [edge_index[1], jnp.full((pad,), NN, jnp.int32)]).reshape(NW, NCH, CH)

    degp = _sc_degree(dst_p, jnp.ones((CH, 8), f32), jnp.zeros((ZR, 8), f32))
    g1, dinv = _tc_first(x_p, W1, degp)
    s1 = _sc_scatter_rows(g1, src_p, dst_p, jnp.zeros((ZR, 64), f32), 64)
    g2 = _tc_fuse(s1, g1, dinv, b1.reshape(1, -1), W2)
    s2 = _sc_scatter_rows(g2, src_p, dst_p, jnp.zeros((ZR, 32), f32), 32)
    g3 = _tc_fuse(s2, g2, dinv, b2.reshape(1, -1), W3)
    s3 = _sc_scatter_rows(g3, src_p, dst_p, jnp.zeros((ZR, 16), f32), 16)
    W4p = jnp.zeros((16, 8), f32).at[:, :4].set(W4)
    b4p = jnp.zeros((8,), f32).at[:4].set(b4)
    Wop = jnp.zeros((8, 1), f32).at[:4].set(Wo)
    g4 = _tc_fuse(s3, g3, dinv, b3.reshape(1, -1), W4p)
    s4 = _sc_scatter_rows(g4, src_p, dst_p, jnp.zeros((ZR, 8), f32), 8)
    return _tc_pool(s4, g4, dinv, b4p.reshape(1, -1),
                    batch.reshape(1, NN), Wop, bo.reshape(1, 1))
